# baseline (speedup 1.0000x reference)
import jax
import jax.numpy as jnp
from jax.experimental import pallas as pl

_HEADS = 4
_DH = 16
_SCALE = 1.0 / (_DH ** 0.5)


def _hgt_kernel(q_ref, k_ref, Ht_ref, ef_ref, packA_ref, packB_ref, out_ref):
    f32 = jnp.float32
    q = q_ref[...]
    k = k_ref[...]
    oc = 64
    m = ef_ref.shape[0]
    neg_inf = float("-inf")

    we2i = packA_ref[0:16, :]
    kqv_w = packA_ref[16:144, :]
    q2_w = packA_ref[144:272, :]
    kmix = packA_ref[272:400, :]
    wk1 = kmix[:, :oc]
    wk2 = kmix[:oc, oc:]
    wv2 = kmix[oc:, oc:]
    oo = packA_ref[400:464, :]
    wo1 = oo[:, :oc]
    wo2 = oo[:, oc:]
    womix = packA_ref[464:528, :]
    wo_ = womix[:, :oc]
    g1c = womix[:, 64:65]
    b1c = womix[:, 65:66]
    g2c = womix[:, 66:67]
    b2c = womix[:, 67:68]
    bffn2c = womix[:, 68:69]
    f1 = packA_ref[528:656, :]
    wffn1 = jnp.concatenate([f1[:64, :], f1[64:, :]], axis=1)
    wffn2 = packB_ref[:, :oc]
    bffn1c = packB_ref[:, 64:65]

    maddT = jnp.where(Ht_ref[...].astype(f32) > 0.0, 0.0, neg_inf)

    hsel = jax.lax.broadcasted_iota(jnp.int32, (_HEADS, 1, oc), 2) // _DH
    hid = jax.lax.broadcasted_iota(jnp.int32, (_HEADS, 1, oc), 0)
    mh3 = (hsel == hid).astype(f32)
    hselr = jax.lax.broadcasted_iota(jnp.int32, (_HEADS, oc, 1), 1) // _DH
    hidr = jax.lax.broadcasted_iota(jnp.int32, (_HEADS, oc, 1), 0)
    mhr = (hselr == hidr).astype(f32)

    ef = ef_ref[...] @ we2i
    khw = (ef @ wk1) * _SCALE

    KQV = k @ kqv_w
    khw4 = khw[None, :, :] * mh3
    khw4p = jnp.concatenate([khw4, jnp.zeros_like(khw4)], axis=2)
    s = jax.lax.dot_general(khw4p.reshape(_HEADS * m, 2 * oc), KQV,
                            (((1,), (1,)), ((), ())))
    s3 = s.reshape(_HEADS, m, -1) + maddT[None, :, :]
    rmax = jnp.maximum(jnp.max(s3, axis=2, keepdims=True), -1e30)
    ex3 = jnp.exp(s3 - rmax)
    rsum = jnp.sum(ex3, axis=2, keepdims=True)
    rinv = jnp.where(rsum > 0.0, 1.0 / rsum, 0.0)
    P = jax.lax.dot_general(ex3.reshape(_HEADS * m, -1), KQV,
                            (((1,), (0,)), ((), ())))
    Vpart = P.reshape(_HEADS, m, 2 * oc)[:, :, oc:]
    he_upd = jnp.sum(Vpart * rinv * mh3, axis=0)
    new_he = he_upd @ wo1

    Q2 = q @ q2_w[:, :oc]
    K2 = new_he @ (wk2 * _SCALE)
    V2T = jax.lax.dot_general(wv2, new_he,
                              (((0,), (1,)), ((), ())))
    K2stack = (K2[None, :, :] * mh3).reshape(_HEADS * m, oc)
    s2 = jax.lax.dot_general(K2stack, Q2,
                             (((1,), (1,)), ((), ())))
    s23 = s2.reshape(_HEADS, m, -1) + maddT[None, :, :]
    cmax = jnp.maximum(jnp.max(s23, axis=1, keepdims=True), -1e30)
    ex2 = jnp.exp(s23 - cmax)
    csum = jnp.sum(ex2, axis=1, keepdims=True)
    rinv2 = jnp.where(csum > 0.0, 1.0 / csum, 0.0)
    a2 = (ex2 * rinv2).reshape(_HEADS * m, -1)
    V2Tstack = jnp.concatenate([V2T * mhr[h] for h in range(_HEADS)],
                               axis=1)
    node_updT = jax.lax.dot_general(V2Tstack, a2,
                                    (((1,), (0,)), ((), ())))

    node_msgT = jax.lax.dot_general(wo2, node_updT,
                                    (((0,), (0,)), ((), ())))
    resT = jax.lax.dot_general(q2_w[:, oc:], q,
                               (((0,), (1,)), ((), ())))
    hhT = jax.lax.dot_general(wo_, node_msgT,
                              (((0,), (0,)), ((), ()))) + resT
    n_inv = 1.0 / hhT.shape[1]
    mu = jnp.sum(hhT, axis=1, keepdims=True) * n_inv
    msq = jnp.sum(hhT * hhT, axis=1, keepdims=True) * n_inv
    sc1 = g1c / jnp.sqrt(msq - mu * mu + 1e-5)
    hhT = hhT * sc1 + (b1c - mu * sc1)
    hT_in = hhT
    tT = jax.lax.dot_general(wffn1, hhT,
                             (((0,), (0,)), ((), ()))) + bffn1c
    tT = 0.5 * tT * (1.0 + jax.lax.erf(tT * (2.0 ** -0.5)))
    hhT = jax.lax.dot_general(wffn2, tT,
                              (((0,), (0,)), ((), ()))) + bffn2c
    hhT = hhT + hT_in
    mu = jnp.sum(hhT, axis=1, keepdims=True) * n_inv
    msq = jnp.sum(hhT * hhT, axis=1, keepdims=True) * n_inv
    sc2 = g2c / jnp.sqrt(msq - mu * mu + 1e-5)
    outT = hhT * sc2 + (b2c - mu * sc2)
    out_ref[...] = outT.T


def kernel(graph, q, k, v, edge_feat, H, W_e2i, W_n2h_q, W_n2h_k, W_n2h_v,
           W_n2h_o, W_h2n_q, W_h2n_k, W_h2n_v, W_h2n_o, W_o, W_ffn1, b_ffn1,
           W_ffn2, b_ffn2, W_res, bn1_g, bn1_b, bn2_g, bn2_b):
    num_nodes = q.shape[0]
    oc = W_n2h_q.shape[1]
    f32 = jnp.float32
    packA = jnp.concatenate([
        W_e2i,
        jnp.concatenate([W_n2h_q, W_n2h_v], axis=1),
        jnp.concatenate([W_h2n_q, W_res], axis=1),
        jnp.concatenate([W_n2h_k,
                         jnp.concatenate([W_h2n_k, W_h2n_v], axis=0)], axis=1),
        jnp.concatenate([W_n2h_o, W_h2n_o], axis=1),
        jnp.concatenate([W_o, bn1_g[:, None], bn1_b[:, None], bn2_g[:, None],
                         bn2_b[:, None], b_ffn2[:, None],
                         jnp.zeros((oc, 59), f32)], axis=1),
        jnp.concatenate([W_ffn1[:, :128], W_ffn1[:, 128:]], axis=0),
    ], axis=0)
    packB = jnp.concatenate([W_ffn2, b_ffn1[:, None],
                             jnp.zeros((4 * oc, 63), f32)], axis=1)
    Ht8 = H.T.astype(jnp.int8)
    return pl.pallas_call(
        _hgt_kernel,
        out_shape=jax.ShapeDtypeStruct((num_nodes, oc), jnp.float32),
    )(q, k, Ht8, edge_feat, packA, packB)
